# Initial kernel scaffold; baseline (speedup 1.0000x reference)
import functools

import jax
import jax.numpy as jnp
from jax import lax
from jax.experimental import pallas as pl
from jax.experimental.pallas import tpu as pltpu
from jax.experimental.pallas import tpu_sc as plsc

VOCAB = 8192
D = 8192
NC = 2
NS = 16
NW = NC * NS

K = 6
NBUF = 2


def _gather_kernel(n_chunks: int, b_per_w: int):
    mesh = plsc.VectorSubcoreMesh(core_axis_name="c", subcore_axis_name="s")

    @functools.partial(
        pl.kernel,
        out_type=jax.ShapeDtypeStruct((NW * b_per_w, D), jnp.float32),
        mesh=mesh,
        scratch_types=[
            pltpu.VMEM((n_chunks, K), jnp.int32),
            pltpu.VMEM((NBUF, K, D), jnp.float32),
            pltpu.SemaphoreType.DMA((NBUF,)),
            pltpu.SemaphoreType.DMA((NBUF,)),
        ],
    )
    def kern(idx_hbm, tab_hbm, out_hbm, idx_v, rows_v, gsem, ssem):
        wid = lax.axis_index("s") * NC + lax.axis_index("c")
        base = wid * b_per_w
        pltpu.sync_copy(idx_hbm.at[wid], idx_v)

        def start_gather(g, slot):
            pltpu.async_copy(
                tab_hbm.at[idx_v.at[g]], rows_v.at[slot], gsem.at[slot]
            )

        def start_store(g, slot):
            pltpu.async_copy(
                rows_v.at[slot], out_hbm.at[pl.ds(base + g * K, K)],
                ssem.at[slot],
            )

        def wait_gather(slot):
            pltpu.make_async_copy(
                tab_hbm.at[idx_v.at[0]], rows_v.at[slot], gsem.at[slot]
            ).wait()

        def wait_store(slot):
            pltpu.make_async_copy(
                rows_v.at[slot], out_hbm.at[pl.ds(base, K)], ssem.at[slot]
            ).wait()

        start_gather(0, 0)

        def body(g, _):
            slot = lax.rem(g, NBUF)
            nslot = lax.rem(g + 1, NBUF)

            @pl.when(g + 1 < n_chunks)
            def _():
                @pl.when(g + 1 >= NBUF)
                def _():
                    wait_store(nslot)

                start_gather(g + 1, nslot)

            wait_gather(slot)
            start_store(g, slot)
            return 0

        lax.fori_loop(0, n_chunks, body, 0)

        def drain(g, _):
            wait_store(lax.rem(g, NBUF))
            return 0

        lax.fori_loop(
            n_chunks - min(NBUF, n_chunks), n_chunks, drain, 0
        )

    return kern


def kernel(x, table):
    b, s = x.shape
    n = b * s
    b_per_w = n // NW
    n_chunks = b_per_w // K
    idx = x.reshape(NW, n_chunks, K).astype(jnp.int32)
    out = _gather_kernel(n_chunks, b_per_w)(idx, table)
    return out.reshape(b, s, D)


# speedup vs baseline: 1.2091x; 1.2091x over previous
import functools

import jax
import jax.numpy as jnp
from jax import lax
from jax.experimental import pallas as pl
from jax.experimental.pallas import tpu as pltpu
from jax.experimental.pallas import tpu_sc as plsc

VOCAB = 8192
D = 8192
NC = 2
NS = 16
NW = NC * NS

K = 5
NBUF = 2


def _gather_kernel(n_chunks: int, b_per_w: int):
    mesh = plsc.VectorSubcoreMesh(
        core_axis_name="c", subcore_axis_name="s",
        num_cores=NC, num_subcores=NS,
    )

    @functools.partial(
        pl.kernel,
        out_type=jax.ShapeDtypeStruct((NW * b_per_w, D), jnp.float32),
        mesh=mesh,
        scratch_types=[
            pltpu.VMEM((n_chunks, K), jnp.int32),
            pltpu.VMEM((NBUF, K, D), jnp.float32),
            pltpu.SemaphoreType.DMA((NBUF,)),
            pltpu.SemaphoreType.DMA((NBUF,)),
        ],
        compiler_params=pltpu.CompilerParams(use_tc_tiling_on_sc=False),
    )
    def kern(idx_hbm, tab_hbm, out_hbm, idx_v, rows_v, gsem, ssem):
        wid = lax.axis_index("s") * NC + lax.axis_index("c")
        base = wid * b_per_w
        pltpu.sync_copy(idx_hbm.at[wid], idx_v)

        def start_gather(g, slot):
            pltpu.async_copy(
                tab_hbm.at[idx_v.at[g]], rows_v.at[slot], gsem.at[slot]
            )

        def start_store(g, slot):
            pltpu.async_copy(
                rows_v.at[slot], out_hbm.at[pl.ds(base + g * K, K)],
                ssem.at[slot],
            )

        def wait_gather(slot):
            pltpu.make_async_copy(
                tab_hbm.at[idx_v.at[0]], rows_v.at[slot], gsem.at[slot]
            ).wait()

        def wait_store(slot):
            pltpu.make_async_copy(
                rows_v.at[slot], out_hbm.at[pl.ds(base, K)], ssem.at[slot]
            ).wait()

        start_gather(0, 0)

        def body(g, _):
            slot = lax.rem(g, NBUF)
            nslot = lax.rem(g + 1, NBUF)

            @pl.when(g + 1 < n_chunks)
            def _():
                @pl.when(g + 1 >= NBUF)
                def _():
                    wait_store(nslot)

                start_gather(g + 1, nslot)

            wait_gather(slot)
            start_store(g, slot)
            return 0

        lax.fori_loop(0, n_chunks, body, 0)

        def drain(g, _):
            wait_store(lax.rem(g, NBUF))
            return 0

        lax.fori_loop(
            n_chunks - min(NBUF, n_chunks), n_chunks, drain, 0
        )

    return kern


def kernel(x, table):
    b, s = x.shape
    n = b * s
    b_per_w = n // NW
    n_chunks = b_per_w // K
    idx = x.reshape(NW, n_chunks, K).astype(jnp.int32)
    out = _gather_kernel(n_chunks, b_per_w)(idx, table)
    return out.reshape(b, s, D)


# speedup vs baseline: 1.2111x; 1.0017x over previous
import functools

import jax
import jax.numpy as jnp
from jax import lax
from jax.experimental import pallas as pl
from jax.experimental.pallas import tpu as pltpu
from jax.experimental.pallas import tpu_sc as plsc

VOCAB = 8192
D = 8192
NC = 2
NS = 16
NW = NC * NS

K = 2
NBUF = 4


def _gather_kernel(n_chunks: int, b_per_w: int):
    mesh = plsc.VectorSubcoreMesh(
        core_axis_name="c", subcore_axis_name="s",
        num_cores=NC, num_subcores=NS,
    )

    @functools.partial(
        pl.kernel,
        out_type=jax.ShapeDtypeStruct((NW * b_per_w, D), jnp.float32),
        mesh=mesh,
        scratch_types=[
            pltpu.VMEM((n_chunks, K), jnp.int32),
            pltpu.VMEM((NBUF, K, D), jnp.float32),
            pltpu.SemaphoreType.DMA((NBUF,)),
            pltpu.SemaphoreType.DMA((NBUF,)),
        ],
        compiler_params=pltpu.CompilerParams(use_tc_tiling_on_sc=False),
    )
    def kern(idx_hbm, tab_hbm, out_hbm, idx_v, rows_v, gsem, ssem):
        wid = lax.axis_index("s") * NC + lax.axis_index("c")
        base = wid * b_per_w
        pltpu.sync_copy(idx_hbm.at[wid], idx_v)

        def start_gather(g, slot):
            pltpu.async_copy(
                tab_hbm.at[idx_v.at[g]], rows_v.at[slot], gsem.at[slot]
            )

        def start_store(g, slot):
            pltpu.async_copy(
                rows_v.at[slot], out_hbm.at[pl.ds(base + g * K, K)],
                ssem.at[slot],
            )

        def wait_gather(slot):
            pltpu.make_async_copy(
                tab_hbm.at[idx_v.at[0]], rows_v.at[slot], gsem.at[slot]
            ).wait()

        def wait_store(slot):
            pltpu.make_async_copy(
                rows_v.at[slot], out_hbm.at[pl.ds(base, K)], ssem.at[slot]
            ).wait()

        for j in range(NBUF - 1):
            start_gather(j, j)

        def body(g, _):
            slot = lax.rem(g, NBUF)
            wait_gather(slot)
            start_store(g, slot)
            ng = g + NBUF - 1

            @pl.when(ng < n_chunks)
            def _():
                pslot = lax.rem(ng, NBUF)
                @pl.when(g >= 1)
                def _():
                    wait_store(pslot)

                start_gather(ng, pslot)

            return 0

        lax.fori_loop(0, n_chunks, body, 0)

        def drain(g, _):
            wait_store(lax.rem(g, NBUF))
            return 0

        lax.fori_loop(n_chunks - NBUF, n_chunks, drain, 0)

    return kern


def kernel(x, table):
    b, s = x.shape
    n = b * s
    b_per_w = n // NW
    n_chunks = b_per_w // K
    idx = x.reshape(NW, n_chunks, K).astype(jnp.int32)
    out = _gather_kernel(n_chunks, b_per_w)(idx, table)
    return out.reshape(b, s, D)


# speedup vs baseline: 3.6613x; 3.0231x over previous
import functools

import jax
import jax.numpy as jnp
from jax import lax
from jax.experimental import pallas as pl
from jax.experimental.pallas import tpu as pltpu
from jax.experimental.pallas import tpu_sc as plsc

VOCAB = 8192
D = 8192
SEQ = 20
LANES = 128
RPR = D // LANES
NC = 2
NS = 16
NW = NC * NS

QUARTS = 4
KR = 8 * (RPR // QUARTS)
NBUF = 5


def _gather_kernel(n_chunks: int, n_batch: int):
    mesh = plsc.VectorSubcoreMesh(
        core_axis_name="c", subcore_axis_name="s",
        num_cores=NC, num_subcores=NS,
    )
    octets_per_w = n_batch // 8 // NW
    chunks_per_w = octets_per_w * SEQ * QUARTS
    arows = n_batch // 8

    @functools.partial(
        pl.kernel,
        out_type=jax.ShapeDtypeStruct(
            (SEQ * arows * RPR * 8, LANES), jnp.float32
        ),
        mesh=mesh,
        scratch_types=[
            pltpu.VMEM((chunks_per_w, KR), jnp.int32),
            pltpu.VMEM((NBUF, KR, LANES), jnp.float32),
            pltpu.SemaphoreType.DMA((NBUF,)),
            pltpu.SemaphoreType.DMA((NBUF,)),
        ],
        compiler_params=pltpu.CompilerParams(use_tc_tiling_on_sc=False),
    )
    def kern(gidx_hbm, tab_hbm, out_hbm, gidx_v, rows_v, gsem, ssem):
        wid = lax.axis_index("s") * NC + lax.axis_index("c")
        cbase = wid * chunks_per_w
        pltpu.sync_copy(gidx_hbm.at[pl.ds(cbase, chunks_per_w)], gidx_v)

        def out_base(g):
            a_local = g // (SEQ * QUARTS)
            rem = lax.rem(g, SEQ * QUARTS)
            s = rem // QUARTS
            q = lax.rem(rem, QUARTS)
            a = wid * octets_per_w + a_local
            return (s * arows + a) * (RPR * 8) + q * KR

        def start_gather(g, slot):
            pltpu.async_copy(
                tab_hbm.at[gidx_v.at[g]], rows_v.at[slot], gsem.at[slot]
            )

        def start_store(g, slot):
            pltpu.async_copy(
                rows_v.at[slot],
                out_hbm.at[pl.ds(out_base(g), KR)],
                ssem.at[slot],
            )

        def wait_gather(slot):
            pltpu.make_async_copy(
                tab_hbm.at[gidx_v.at[0]], rows_v.at[slot], gsem.at[slot]
            ).wait()

        def wait_store(slot):
            pltpu.make_async_copy(
                rows_v.at[slot],
                out_hbm.at[pl.ds(0, KR)],
                ssem.at[slot],
            ).wait()

        for j in range(NBUF - 1):
            start_gather(j, j)

        def body(g, _):
            slot = lax.rem(g, NBUF)
            wait_gather(slot)
            start_store(g, slot)
            ng = g + NBUF - 1

            @pl.when(ng < chunks_per_w)
            def _():
                pslot = lax.rem(ng, NBUF)
                @pl.when(g >= 1)
                def _():
                    wait_store(pslot)

                start_gather(ng, pslot)

            return 0

        lax.fori_loop(0, chunks_per_w, body, 0)

        def drain(g, _):
            wait_store(lax.rem(g, NBUF))
            return 0

        lax.fori_loop(chunks_per_w - NBUF, chunks_per_w, drain, 0)

    return kern


def kernel(x, table):
    b, s = x.shape
    n = b * s
    n_chunks = n // 8 * QUARTS

    xi = x.astype(jnp.int32)
    w = jnp.arange(NW, dtype=jnp.int32)
    a_local = jnp.arange(b // 8 // NW, dtype=jnp.int32)
    sq = jnp.arange(s, dtype=jnp.int32)
    q = jnp.arange(QUARTS, dtype=jnp.int32)
    c_local = jnp.arange(RPR // QUARTS, dtype=jnp.int32)
    r = jnp.arange(8, dtype=jnp.int32)

    octet = (w[:, None] * (b // 8 // NW) + a_local[None, :])
    batch = octet[..., None] * 8 + r
    xv = xi[batch][:, :, :, :]
    xv = xv.transpose(0, 1, 3, 2)
    base_run = (xv // 8) * (RPR * 8) + (xv % 8)
    cc = q[:, None] * (RPR // QUARTS) + c_local[None, :]
    gidx = (
        base_run[:, :, :, None, None, :]
        + 8 * cc[None, None, None, :, :, None]
    )
    gidx = gidx.reshape(n_chunks, KR)

    tabv = (
        table.reshape(VOCAB // 8, 8, D // LANES, LANES)
        .transpose(0, 2, 1, 3)
        .reshape(VOCAB * D // LANES, LANES)
    )

    out = _gather_kernel(n_chunks, b)(gidx, tabv)
    return (
        out.reshape(s, b // 8, D // LANES, 8, LANES)
        .transpose(1, 3, 0, 2, 4)
        .reshape(b, s, D)
    )


# speedup vs baseline: 3.7421x; 1.0221x over previous
import functools

import jax
import jax.numpy as jnp
from jax import lax
from jax.experimental import pallas as pl
from jax.experimental.pallas import tpu as pltpu
from jax.experimental.pallas import tpu_sc as plsc

VOCAB = 8192
D = 8192
SEQ = 20
LANES = 128
RPR = D // LANES
NC = 2
NS = 16
NW = NC * NS

QUARTS = 4
KR = 8 * (RPR // QUARTS)
NBUF = 5


def _gather_kernel(n_chunks: int, n_batch: int):
    mesh = plsc.VectorSubcoreMesh(
        core_axis_name="c", subcore_axis_name="s",
        num_cores=NC, num_subcores=NS,
    )
    octets_per_w = n_batch // 8 // NW
    chunks_per_w = octets_per_w * SEQ * QUARTS
    arows = n_batch // 8

    @functools.partial(
        pl.kernel,
        out_type=jax.ShapeDtypeStruct(
            (SEQ * arows * RPR * 8, LANES), jnp.float32
        ),
        mesh=mesh,
        scratch_types=[
            pltpu.VMEM((chunks_per_w, KR), jnp.int32),
            pltpu.VMEM((NBUF, KR, LANES), jnp.float32),
            pltpu.SemaphoreType.DMA((NBUF,)),
            pltpu.SemaphoreType.DMA((NBUF,)),
        ],
        compiler_params=pltpu.CompilerParams(use_tc_tiling_on_sc=False),
    )
    def kern(gidx_hbm, tab_hbm, out_hbm, gidx_v, rows_v, gsem, ssem):
        wid = lax.axis_index("s") * NC + lax.axis_index("c")
        cbase = wid * chunks_per_w
        pltpu.sync_copy(gidx_hbm.at[pl.ds(cbase, chunks_per_w)], gidx_v)

        wbase = wid * octets_per_w * RPR * 8
        m_per_s = octets_per_w * QUARTS

        def out_base(g):
            s = g // m_per_s
            m = lax.rem(g, m_per_s)
            return s * (arows * RPR * 8) + wbase + m * KR

        def start_gather(g, slot):
            pltpu.async_copy(
                tab_hbm.at[gidx_v.at[g]], rows_v.at[slot], gsem.at[slot]
            )

        def start_store(g, slot):
            pltpu.async_copy(
                rows_v.at[slot],
                out_hbm.at[pl.ds(out_base(g), KR)],
                ssem.at[slot],
            )

        def wait_gather(slot):
            pltpu.make_async_copy(
                tab_hbm.at[gidx_v.at[0]], rows_v.at[slot], gsem.at[slot]
            ).wait()

        def wait_store(slot):
            pltpu.make_async_copy(
                rows_v.at[slot],
                out_hbm.at[pl.ds(0, KR)],
                ssem.at[slot],
            ).wait()

        for j in range(NBUF - 1):
            start_gather(j, j)

        def body(g, _):
            slot = lax.rem(g, NBUF)
            wait_gather(slot)
            start_store(g, slot)
            ng = g + NBUF - 1

            @pl.when(ng < chunks_per_w)
            def _():
                pslot = lax.rem(ng, NBUF)
                @pl.when(g >= 1)
                def _():
                    wait_store(pslot)

                start_gather(ng, pslot)

            return 0

        lax.fori_loop(0, chunks_per_w, body, 0)

        def drain(g, _):
            wait_store(lax.rem(g, NBUF))
            return 0

        lax.fori_loop(chunks_per_w - NBUF, chunks_per_w, drain, 0)

    return kern


def kernel(x, table):
    b, s = x.shape
    n = b * s
    n_chunks = n // 8 * QUARTS

    xi = x.astype(jnp.int32)
    w = jnp.arange(NW, dtype=jnp.int32)
    a_local = jnp.arange(b // 8 // NW, dtype=jnp.int32)
    sq = jnp.arange(s, dtype=jnp.int32)
    q = jnp.arange(QUARTS, dtype=jnp.int32)
    c_local = jnp.arange(RPR // QUARTS, dtype=jnp.int32)
    r = jnp.arange(8, dtype=jnp.int32)

    octet = (w[:, None] * (b // 8 // NW) + a_local[None, :])
    batch = octet[..., None] * 8 + r
    xv = xi[batch][:, :, :, :]
    xv = xv.transpose(0, 3, 1, 2)
    base_run = (xv // 8) * (RPR * 8) + (xv % 8)
    cc = q[:, None] * (RPR // QUARTS) + c_local[None, :]
    gidx = (
        base_run[:, :, :, None, None, :]
        + 8 * cc[None, None, None, :, :, None]
    )
    gidx = gidx.reshape(n_chunks, KR)

    tabv = (
        table.reshape(VOCAB // 8, 8, D // LANES, LANES)
        .transpose(0, 2, 1, 3)
        .reshape(VOCAB * D // LANES, LANES)
    )

    out = _gather_kernel(n_chunks, b)(gidx, tabv)
    return (
        out.reshape(s, b // 8, D // LANES, 8, LANES)
        .transpose(1, 3, 0, 2, 4)
        .reshape(b, s, D)
    )


# speedup vs baseline: 4.2653x; 1.1398x over previous
import functools

import jax
import jax.numpy as jnp
from jax import lax
from jax.experimental import pallas as pl
from jax.experimental.pallas import tpu as pltpu
from jax.experimental.pallas import tpu_sc as plsc

VOCAB = 8192
D = 8192
SEQ = 20
LANES = 128
RPR = D // LANES
NC = 2
NS = 16
NW = NC * NS

NBUF = 10


def _gather_kernel(n_batch: int):
    mesh = plsc.VectorSubcoreMesh(
        core_axis_name="c", subcore_axis_name="s",
        num_cores=NC, num_subcores=NS,
    )
    b_per_w = n_batch // NW
    n_per_w = b_per_w * SEQ
    arows = n_batch // 8

    @functools.partial(
        pl.kernel,
        out_type=jax.ShapeDtypeStruct((SEQ * arows * 512, LANES), jnp.float32),
        mesh=mesh,
        scratch_types=[
            pltpu.VMEM((n_per_w, RPR), jnp.int32),
            pltpu.VMEM((NBUF, RPR), jnp.int32),
            pltpu.VMEM((NBUF, RPR, LANES), jnp.float32),
            pltpu.SemaphoreType.DMA((NBUF,)),
            pltpu.SemaphoreType.DMA((NBUF,)),
        ],
        compiler_params=pltpu.CompilerParams(use_tc_tiling_on_sc=False),
    )
    def kern(gidx_hbm, tab_hbm, out_hbm, gidx_v, sidx_v, rows_v, gsem, ssem):
        wid = lax.axis_index("s") * NC + lax.axis_index("c")
        cbase = wid * n_per_w
        wb64 = wid * b_per_w * RPR
        pltpu.sync_copy(gidx_hbm.at[pl.ds(cbase, n_per_w)], gidx_v)

        iota16 = lax.iota(jnp.int32, 16)

        def start_gather(g, slot):
            pltpu.async_copy(
                tab_hbm.at[gidx_v.at[g]], rows_v.at[slot], gsem.at[slot]
            )

        def fill_sidx(g, slot):
            s = g // SEQ_BLK
            b_local = lax.rem(g, SEQ_BLK)
            sbase = (
                s * (arows * 512)
                + wb64
                + (b_local // 8) * 512
                + lax.rem(b_local, 8)
            )
            vec = jnp.full((16,), sbase, jnp.int32) + 8 * iota16
            for j in range(RPR // 16):
                sidx_v[slot, pl.ds(j * 16, 16)] = vec + j * 128

        def start_store(g, slot):
            pltpu.async_copy(
                rows_v.at[slot], out_hbm.at[sidx_v.at[slot]], ssem.at[slot]
            )

        def wait_gather(slot):
            pltpu.make_async_copy(
                tab_hbm.at[gidx_v.at[0]], rows_v.at[slot], gsem.at[slot]
            ).wait()

        def wait_store(slot):
            pltpu.make_async_copy(
                rows_v.at[slot], out_hbm.at[sidx_v.at[0]], ssem.at[slot]
            ).wait()

        for j in range(NBUF - 1):
            start_gather(j, j)

        def body(g, _):
            slot = lax.rem(g, NBUF)
            wait_gather(slot)
            fill_sidx(g, slot)
            start_store(g, slot)
            ng = g + NBUF - 1

            @pl.when(ng < n_per_w)
            def _():
                pslot = lax.rem(ng, NBUF)
                @pl.when(g >= 1)
                def _():
                    wait_store(pslot)

                start_gather(ng, pslot)

            return 0

        lax.fori_loop(0, n_per_w, body, 0)

        def drain(g, _):
            wait_store(lax.rem(g, NBUF))
            return 0

        lax.fori_loop(n_per_w - NBUF, n_per_w, drain, 0)

    return kern


SEQ_BLK = 32


def kernel(x, table):
    b, s = x.shape

    xi = x.astype(jnp.int32)
    xv = xi.reshape(NW, b // NW, s).transpose(0, 2, 1)
    base_run = (xv // 8) * (RPR * 8) + (xv % 8)
    c64 = 8 * jnp.arange(RPR, dtype=jnp.int32)
    gidx = base_run[..., None] + c64
    gidx = gidx.reshape(NW * b // NW * s, RPR)

    tabv = (
        table.reshape(VOCAB // 8, 8, D // LANES, LANES)
        .transpose(0, 2, 1, 3)
        .reshape(VOCAB * D // LANES, LANES)
    )

    out = _gather_kernel(b)(gidx, tabv)
    return (
        out.reshape(s, b // 8, D // LANES, 8, LANES)
        .transpose(1, 3, 0, 2, 4)
        .reshape(b, s, D)
    )
